# baseline (speedup 1.0000x reference)
import functools

import jax
import jax.numpy as jnp
from jax import lax
from jax.experimental import pallas as pl
from jax.experimental.pallas import tpu as pltpu
from jax.experimental.pallas import tpu_sc as plsc


def _sc_body(N, C, H, W, P, WPI, PW, CH, NCH,
             xyz_hbm, table_hbm, depth_hbm, outf_hbm, outz_hbm,
             depth_v, xyz_v, idx_v, rows_v, wx_v, wy_v, dz_v, out_v,
             sem_in, sem_g, sem_out):
    cid = lax.axis_index("c")
    sid = lax.axis_index("s")
    nc = lax.axis_size("c")
    wid = sid * nc + cid
    n = wid // WPI
    part = wid % WPI
    pbase = part * PW
    nbase = n * (H * W)

    pltpu.sync_copy(depth_hbm.at[n], depth_v)

    def in_src(i):
        return xyz_hbm.at[n, :, pl.ds(pbase + i * CH, CH)]

    def fire_in(b, i):
        pltpu.async_copy(in_src(i), xyz_v.at[b], sem_in.at[b])

    def wait_in(b, i):
        pltpu.make_async_copy(in_src(i), xyz_v.at[b], sem_in.at[b]).wait()

    def fire_gather(b):
        for t in range(4):
            pltpu.async_copy(table_hbm.at[idx_v.at[b, t]], rows_v.at[b, t],
                             sem_g.at[b])

    def wait_gather(b):
        for t in range(4):
            pltpu.make_async_copy(table_hbm.at[idx_v.at[b, t]],
                                  rows_v.at[b, t], sem_g.at[b]).wait()

    def out_dsts(b, i):
        off = pbase + i * CH
        pb = off // CH
        return ((out_v.at[b], outf_hbm.at[n, :, pb]),
                (dz_v.at[b], outz_hbm.at[n, pl.ds(off, CH)]))

    def fire_out(b, i):
        for src, dst in out_dsts(b, i):
            pltpu.async_copy(src, dst, sem_out.at[b])

    def wait_out(b, i):
        for src, dst in out_dsts(b, i):
            pltpu.make_async_copy(src, dst, sem_out.at[b]).wait()

    def stage(b):
        for v in range(CH // 16):
            sl = pl.ds(v * 16, 16)
            gx = (xyz_v[b, 0, sl] + 1.0) * ((W - 1) * 0.5)
            gy = (xyz_v[b, 1, sl] + 1.0) * ((H - 1) * 0.5)
            x0 = jnp.minimum(jnp.maximum(gx.astype(jnp.int32), 0), W - 2)
            y0 = jnp.minimum(jnp.maximum(gy.astype(jnp.int32), 0), H - 2)
            wx = gx - x0.astype(jnp.float32)
            wy = gy - y0.astype(jnp.float32)
            r00 = y0 * W + x0
            g00 = r00 + nbase
            idx_v[b, 0, sl] = g00
            idx_v[b, 1, sl] = g00 + 1
            idx_v[b, 2, sl] = g00 + W
            idx_v[b, 3, sl] = g00 + (W + 1)
            x1 = x0 + 1
            d00 = plsc.load_gather(depth_v, [y0, x0])
            d01 = plsc.load_gather(depth_v, [y0, x1])
            d10 = plsc.load_gather(depth_v, [y0 + 1, x0])
            d11 = plsc.load_gather(depth_v, [y0 + 1, x1])
            dx0 = d00 + wx * (d01 - d00)
            dx1 = d10 + wx * (d11 - d10)
            dd = dx0 + wy * (dx1 - dx0)
            df = xyz_v[b, 2, sl] - dd
            dz_v[b, sl] = jnp.exp(-200.0 * df * df)
            wx_v[b, sl] = wx
            wy_v[b, sl] = wy

    def blend(b):
        @plsc.parallel_loop(0, CH, 16)
        def _pt(p0):
            wxv = wx_v[b, pl.ds(p0, 16)]
            wyv = wy_v[b, pl.ds(p0, 16)]
            pidx = p0 + lax.broadcasted_iota(jnp.int32, (16,), 0)
            for c in range(C):
                cidx = jnp.full((16,), c, jnp.int32)
                v00 = plsc.load_gather(rows_v.at[b, 0], [pidx, cidx])
                v01 = plsc.load_gather(rows_v.at[b, 1], [pidx, cidx])
                v10 = plsc.load_gather(rows_v.at[b, 2], [pidx, cidx])
                v11 = plsc.load_gather(rows_v.at[b, 3], [pidx, cidx])
                a = v00 + wxv * (v01 - v00)
                bb = v10 + wxv * (v11 - v10)
                out_v[b, c // 8, c % 8, pl.ds(p0, 16)] = a + wyv * (bb - a)

    fire_in(0, 0)
    fire_in(1, 1)

    def pair_body(k, _):
        for b in (0, 1):
            o = 1 - b
            i = 2 * k + b

            @pl.when(i >= 2)
            def _wait_out_prev():
                wait_out(b, i - 2)

            wait_in(b, i)
            stage(b)
            fire_gather(b)

            @pl.when(i + 2 < NCH)
            def _prefetch():
                fire_in(b, i + 2)

            @pl.when(i >= 1)
            def _blend_prev():
                wait_gather(o)
                blend(o)
                fire_out(o, i - 1)

        return 0

    lax.fori_loop(0, NCH // 2, pair_body, 0)

    last = NCH - 1
    lb = last % 2
    wait_gather(lb)
    blend(lb)
    fire_out(lb, last)
    wait_out(1 - lb, last - 1)
    wait_out(lb, last)


def kernel(rgbd_feats, rgbs, depths, proj_xy, proj_z, num_views):
    del rgbs, num_views
    N, C, H, W = rgbd_feats.shape
    B = proj_xy.shape[0]
    P = proj_xy.shape[2] * proj_xy.shape[3] * proj_xy.shape[4]
    assert N == B * proj_xy.shape[1]

    info = plsc.get_sparse_core_info()
    NW = info.num_cores * info.num_subcores
    WPI = NW // N
    PW = P // WPI
    CH = 128
    NCH = PW // CH
    assert P % WPI == 0 and PW % CH == 0 and NCH % 2 == 0

    table = jnp.transpose(rgbd_feats.reshape(N, C, H * W),
                          (0, 2, 1)).reshape(N * H * W, C)
    depth_t = depths.reshape(N, H, W)
    xyz = jnp.stack(
        [proj_xy[..., 0].reshape(N, P),
         proj_xy[..., 1].reshape(N, P),
         proj_z.reshape(N, P)], axis=1)

    mesh = plsc.VectorSubcoreMesh(core_axis_name="c", subcore_axis_name="s")
    body = functools.partial(_sc_body, N, C, H, W, P, WPI, PW, CH, NCH)
    outf, outz = pl.kernel(
        body,
        out_type=(
            jax.ShapeDtypeStruct((N, C // 8, P // CH, 8, CH), jnp.float32),
            jax.ShapeDtypeStruct((N, P), jnp.float32),
        ),
        mesh=mesh,
        scratch_types=[
            pltpu.VMEM((H, W), jnp.float32),
            pltpu.VMEM((2, 3, CH), jnp.float32),
            pltpu.VMEM((2, 4, CH), jnp.int32),
            pltpu.VMEM((2, 4, CH, C), jnp.float32),
            pltpu.VMEM((2, CH), jnp.float32),
            pltpu.VMEM((2, CH), jnp.float32),
            pltpu.VMEM((2, CH), jnp.float32),
            pltpu.VMEM((2, C // 8, 8, CH), jnp.float32),
            pltpu.SemaphoreType.DMA((2,)),
            pltpu.SemaphoreType.DMA((2,)),
            pltpu.SemaphoreType.DMA((2,)),
        ],
        compiler_params=pltpu.CompilerParams(needs_layout_passes=False,
                                             use_tc_tiling_on_sc=False),
        name="mv_grid_sample_sc",
    )(xyz, table, depth_t)
    outf = outf.transpose(0, 2, 4, 1, 3).reshape(N, P, C)
    return outf, outz.reshape(N, P, 1)


# speedup vs baseline: 2.1182x; 2.1182x over previous
import functools

import jax
import jax.numpy as jnp
from jax import lax
from jax.experimental import pallas as pl
from jax.experimental.pallas import tpu as pltpu
from jax.experimental.pallas import tpu_sc as plsc


def _sc_body(N, C, H, W, P, WPI, PW, CH, NCH,
             xyz_hbm, table_hbm, depth_hbm, outf_hbm, outz_hbm,
             depth_v, xyz_v, idx_v, rows_v, wx_v, wy_v, dz_v, out_v,
             sem_in, sem_g, sem_out):
    cid = lax.axis_index("c")
    sid = lax.axis_index("s")
    nc = lax.axis_size("c")
    wid = sid * nc + cid
    n = wid // WPI
    part = wid % WPI
    pbase = part * PW
    nbase = n * (H * W)

    pltpu.sync_copy(depth_hbm.at[n], depth_v)

    def in_src(i):
        return xyz_hbm.at[n, :, pl.ds(pbase + i * CH, CH)]

    def fire_in(b, i):
        pltpu.async_copy(in_src(i), xyz_v.at[b], sem_in.at[b])

    def wait_in(b, i):
        pltpu.make_async_copy(in_src(i), xyz_v.at[b], sem_in.at[b]).wait()

    def fire_gather(b):
        for t in range(4):
            pltpu.async_copy(table_hbm.at[idx_v.at[b, t]], rows_v.at[b, t],
                             sem_g.at[b])

    def wait_gather(b):
        for t in range(4):
            pltpu.make_async_copy(table_hbm.at[idx_v.at[b, t]],
                                  rows_v.at[b, t], sem_g.at[b]).wait()

    def out_dsts(b, i):
        off = pbase + i * CH
        pb = off // CH
        return ((out_v.at[b], outf_hbm.at[n, :, pb]),
                (dz_v.at[b], outz_hbm.at[n, pl.ds(off, CH)]))

    def fire_out(b, i):
        for src, dst in out_dsts(b, i):
            pltpu.async_copy(src, dst, sem_out.at[b])

    def wait_out(b, i):
        for src, dst in out_dsts(b, i):
            pltpu.make_async_copy(src, dst, sem_out.at[b]).wait()

    def stage(b):
        for v in range(CH // 16):
            sl = pl.ds(v * 16, 16)
            gx = (xyz_v[b, 0, sl] + 1.0) * ((W - 1) * 0.5)
            gy = (xyz_v[b, 1, sl] + 1.0) * ((H - 1) * 0.5)
            x0 = jnp.minimum(jnp.maximum(gx.astype(jnp.int32), 0), W - 2)
            y0 = jnp.minimum(jnp.maximum(gy.astype(jnp.int32), 0), H - 2)
            wx = gx - x0.astype(jnp.float32)
            wy = gy - y0.astype(jnp.float32)
            r00 = y0 * W + x0
            g00 = r00 + nbase
            idx_v[b, 0, sl] = g00
            idx_v[b, 1, sl] = g00 + 1
            idx_v[b, 2, sl] = g00 + W
            idx_v[b, 3, sl] = g00 + (W + 1)
            x1 = x0 + 1
            d00 = plsc.load_gather(depth_v, [y0, x0])
            d01 = plsc.load_gather(depth_v, [y0, x1])
            d10 = plsc.load_gather(depth_v, [y0 + 1, x0])
            d11 = plsc.load_gather(depth_v, [y0 + 1, x1])
            dx0 = d00 + wx * (d01 - d00)
            dx1 = d10 + wx * (d11 - d10)
            dd = dx0 + wy * (dx1 - dx0)
            df = xyz_v[b, 2, sl] - dd
            dz_v[b, sl] = jnp.exp(-200.0 * df * df)
            wx_v[b, sl] = wx
            wy_v[b, sl] = wy

    def blend(b):
        @plsc.parallel_loop(0, CH, 16)
        def _pt(p0):
            wxv = wx_v[b, pl.ds(p0, 16)]
            wyv = wy_v[b, pl.ds(p0, 16)]
            lane = lax.broadcasted_iota(jnp.int32, (16,), 0)
            pidx = p0 + lane
            for c in range(C):
                cvec = (c + lane) & (C - 1)
                v00 = plsc.load_gather(rows_v.at[b, 0], [pidx, cvec])
                v01 = plsc.load_gather(rows_v.at[b, 1], [pidx, cvec])
                v10 = plsc.load_gather(rows_v.at[b, 2], [pidx, cvec])
                v11 = plsc.load_gather(rows_v.at[b, 3], [pidx, cvec])
                a = v00 + wxv * (v01 - v00)
                bb = v10 + wxv * (v11 - v10)
                res = a + wyv * (bb - a)
                plsc.store_scatter(out_v.at[b],
                                   [cvec >> 3, cvec & 7, pidx], res)

    fire_in(0, 0)
    fire_in(1, 1)

    def pair_body(k, _):
        for b in (0, 1):
            o = 1 - b
            i = 2 * k + b

            @pl.when(i >= 2)
            def _wait_out_prev():
                wait_out(b, i - 2)

            wait_in(b, i)
            stage(b)
            fire_gather(b)

            @pl.when(i + 2 < NCH)
            def _prefetch():
                fire_in(b, i + 2)

            @pl.when(i >= 1)
            def _blend_prev():
                wait_gather(o)
                blend(o)
                fire_out(o, i - 1)

        return 0

    lax.fori_loop(0, NCH // 2, pair_body, 0)

    last = NCH - 1
    lb = last % 2
    wait_gather(lb)
    blend(lb)
    fire_out(lb, last)
    wait_out(1 - lb, last - 1)
    wait_out(lb, last)


def kernel(rgbd_feats, rgbs, depths, proj_xy, proj_z, num_views):
    del rgbs, num_views
    N, C, H, W = rgbd_feats.shape
    B = proj_xy.shape[0]
    P = proj_xy.shape[2] * proj_xy.shape[3] * proj_xy.shape[4]
    assert N == B * proj_xy.shape[1]

    info = plsc.get_sparse_core_info()
    NW = info.num_cores * info.num_subcores
    WPI = NW // N
    PW = P // WPI
    CH = 128
    NCH = PW // CH
    assert P % WPI == 0 and PW % CH == 0 and NCH % 2 == 0

    table = jnp.transpose(rgbd_feats.reshape(N, C, H * W),
                          (0, 2, 1)).reshape(N * H * W, C)
    depth_t = depths.reshape(N, H, W)
    xyz = jnp.stack(
        [proj_xy[..., 0].reshape(N, P),
         proj_xy[..., 1].reshape(N, P),
         proj_z.reshape(N, P)], axis=1)

    mesh = plsc.VectorSubcoreMesh(core_axis_name="c", subcore_axis_name="s")
    body = functools.partial(_sc_body, N, C, H, W, P, WPI, PW, CH, NCH)
    outf, outz = pl.kernel(
        body,
        out_type=(
            jax.ShapeDtypeStruct((N, C // 8, P // CH, 8, CH), jnp.float32),
            jax.ShapeDtypeStruct((N, P), jnp.float32),
        ),
        mesh=mesh,
        scratch_types=[
            pltpu.VMEM((H, W), jnp.float32),
            pltpu.VMEM((2, 3, CH), jnp.float32),
            pltpu.VMEM((2, 4, CH), jnp.int32),
            pltpu.VMEM((2, 4, CH, C), jnp.float32),
            pltpu.VMEM((2, CH), jnp.float32),
            pltpu.VMEM((2, CH), jnp.float32),
            pltpu.VMEM((2, CH), jnp.float32),
            pltpu.VMEM((2, C // 8, 8, CH), jnp.float32),
            pltpu.SemaphoreType.DMA((2,)),
            pltpu.SemaphoreType.DMA((2,)),
            pltpu.SemaphoreType.DMA((2,)),
        ],
        compiler_params=pltpu.CompilerParams(needs_layout_passes=False,
                                             use_tc_tiling_on_sc=False),
        name="mv_grid_sample_sc",
    )(xyz, table, depth_t)
    outf = outf.transpose(0, 2, 4, 1, 3).reshape(N, P, C)
    return outf, outz.reshape(N, P, 1)


# speedup vs baseline: 2.3381x; 1.1038x over previous
import functools

import jax
import jax.numpy as jnp
from jax import lax
from jax.experimental import pallas as pl
from jax.experimental.pallas import tpu as pltpu
from jax.experimental.pallas import tpu_sc as plsc


def _sc_body(N, C, H, W, P, WPI, PW, CH, NCH,
             x_hbm, y_hbm, z_hbm, table_hbm, depth_hbm, outf_hbm, outz_hbm,
             depth_v, x_v, y_v, z_v, idx_v, rows_v, wx_v, wy_v, dz_v, out_v,
             sem_in, sem_g, sem_out):
    cid = lax.axis_index("c")
    sid = lax.axis_index("s")
    nc = lax.axis_size("c")
    wid = sid * nc + cid
    n = wid // WPI
    part = wid % WPI
    pbase = part * PW
    nbase = n * (H * W)

    pltpu.sync_copy(depth_hbm.at[n], depth_v)

    def in_pairs(b, i):
        sl = pl.ds(pbase + i * CH, CH)
        return ((x_hbm.at[n, sl], x_v.at[b]),
                (y_hbm.at[n, sl], y_v.at[b]),
                (z_hbm.at[n, sl], z_v.at[b]))

    def fire_in(b, i):
        for src, dst in in_pairs(b, i):
            pltpu.async_copy(src, dst, sem_in.at[b])

    def wait_in(b, i):
        for src, dst in in_pairs(b, i):
            pltpu.make_async_copy(src, dst, sem_in.at[b]).wait()

    def fire_gather(b):
        for t in range(4):
            pltpu.async_copy(table_hbm.at[idx_v.at[b, t]], rows_v.at[b, t],
                             sem_g.at[b])

    def wait_gather(b):
        for t in range(4):
            pltpu.make_async_copy(table_hbm.at[idx_v.at[b, t]],
                                  rows_v.at[b, t], sem_g.at[b]).wait()

    def out_dsts(b, i):
        off = pbase + i * CH
        pb = off // CH
        return ((out_v.at[b], outf_hbm.at[n, :, pb]),
                (dz_v.at[b], outz_hbm.at[n, pl.ds(off, CH)]))

    def fire_out(b, i):
        for src, dst in out_dsts(b, i):
            pltpu.async_copy(src, dst, sem_out.at[b])

    def wait_out(b, i):
        for src, dst in out_dsts(b, i):
            pltpu.make_async_copy(src, dst, sem_out.at[b]).wait()

    def stage(b):
        for v in range(CH // 16):
            sl = pl.ds(v * 16, 16)
            gx = (x_v[b, sl] + 1.0) * ((W - 1) * 0.5)
            gy = (y_v[b, sl] + 1.0) * ((H - 1) * 0.5)
            x0 = jnp.minimum(jnp.maximum(gx.astype(jnp.int32), 0), W - 2)
            y0 = jnp.minimum(jnp.maximum(gy.astype(jnp.int32), 0), H - 2)
            wx = gx - x0.astype(jnp.float32)
            wy = gy - y0.astype(jnp.float32)
            r00 = y0 * W + x0
            g00 = r00 + nbase
            idx_v[b, 0, sl] = g00
            idx_v[b, 1, sl] = g00 + 1
            idx_v[b, 2, sl] = g00 + W
            idx_v[b, 3, sl] = g00 + (W + 1)
            x1 = x0 + 1
            d00 = plsc.load_gather(depth_v, [y0, x0])
            d01 = plsc.load_gather(depth_v, [y0, x1])
            d10 = plsc.load_gather(depth_v, [y0 + 1, x0])
            d11 = plsc.load_gather(depth_v, [y0 + 1, x1])
            dx0 = d00 + wx * (d01 - d00)
            dx1 = d10 + wx * (d11 - d10)
            dd = dx0 + wy * (dx1 - dx0)
            df = z_v[b, sl] - dd
            dz_v[b, sl] = jnp.exp(-200.0 * df * df)
            wx_v[b, sl] = wx
            wy_v[b, sl] = wy

    lane = lax.broadcasted_iota(jnp.int32, (16,), 0)
    cvecs = [(c + lane) & (C - 1) for c in range(C)]
    cbs = [cv >> 3 for cv in cvecs]
    cis = [cv & 7 for cv in cvecs]

    def blend(b):
        @plsc.parallel_loop(0, CH, 16)
        def _pt(p0):
            wxv = wx_v[b, pl.ds(p0, 16)]
            wyv = wy_v[b, pl.ds(p0, 16)]
            pidx = p0 + lane
            for c in range(C):
                v00 = plsc.load_gather(rows_v.at[b, 0], [pidx, cvecs[c]])
                v01 = plsc.load_gather(rows_v.at[b, 1], [pidx, cvecs[c]])
                v10 = plsc.load_gather(rows_v.at[b, 2], [pidx, cvecs[c]])
                v11 = plsc.load_gather(rows_v.at[b, 3], [pidx, cvecs[c]])
                a = v00 + wxv * (v01 - v00)
                bb = v10 + wxv * (v11 - v10)
                res = a + wyv * (bb - a)
                plsc.store_scatter(out_v.at[b], [cbs[c], cis[c], pidx], res)

    fire_in(0, 0)
    fire_in(1, 1)

    def pair_body(k, _):
        for b in (0, 1):
            o = 1 - b
            i = 2 * k + b

            @pl.when(i >= 2)
            def _wait_out_prev():
                wait_out(b, i - 2)

            wait_in(b, i)
            stage(b)
            fire_gather(b)

            @pl.when(i + 2 < NCH)
            def _prefetch():
                fire_in(b, i + 2)

            @pl.when(i >= 1)
            def _blend_prev():
                wait_gather(o)
                blend(o)
                fire_out(o, i - 1)

        return 0

    lax.fori_loop(0, NCH // 2, pair_body, 0)

    last = NCH - 1
    lb = last % 2
    wait_gather(lb)
    blend(lb)
    fire_out(lb, last)
    wait_out(1 - lb, last - 1)
    wait_out(lb, last)


def kernel(rgbd_feats, rgbs, depths, proj_xy, proj_z, num_views):
    del rgbs, num_views
    N, C, H, W = rgbd_feats.shape
    B = proj_xy.shape[0]
    P = proj_xy.shape[2] * proj_xy.shape[3] * proj_xy.shape[4]
    assert N == B * proj_xy.shape[1]

    info = plsc.get_sparse_core_info()
    NW = info.num_cores * info.num_subcores
    WPI = NW // N
    PW = P // WPI
    CH = 128
    NCH = PW // CH
    assert P % WPI == 0 and PW % CH == 0 and NCH % 2 == 0

    def _tr_body(in_ref, out_ref, tmp_ref):
        rows = W * C // 128
        for yy in range(8):
            tmp_ref[...] = jnp.transpose(in_ref[:, yy, :], (1, 0))
            for k in range(4):
                sub = tmp_ref[pl.Slice(k, W // 4, 4), :]
                out_ref[yy * rows:(yy + 1) * rows, k * C:(k + 1) * C] = sub

    table128 = pl.pallas_call(
        _tr_body,
        grid=(N, H // 8),
        in_specs=[pl.BlockSpec((C, 8, W), lambda n, y: (n, y, 0))],
        out_specs=pl.BlockSpec((8 * W * C // 128, 128),
                               lambda n, y: (n * H + y, 0)),
        out_shape=jax.ShapeDtypeStruct((N * H * W * C // 128, 128),
                                       jnp.float32),
        scratch_shapes=[pltpu.VMEM((W, C), jnp.float32)],
    )(rgbd_feats.reshape(N * C, H, W))
    table = table128.reshape(N * H * W, C)

    depth_t = depths.reshape(N, H, W)
    x = proj_xy[..., 0].reshape(N, P)
    y = proj_xy[..., 1].reshape(N, P)
    z = proj_z.reshape(N, P)

    mesh = plsc.VectorSubcoreMesh(core_axis_name="c", subcore_axis_name="s")
    body = functools.partial(_sc_body, N, C, H, W, P, WPI, PW, CH, NCH)
    outf, outz = pl.kernel(
        body,
        out_type=(
            jax.ShapeDtypeStruct((N, C // 8, P // CH, 8, CH), jnp.float32),
            jax.ShapeDtypeStruct((N, P), jnp.float32),
        ),
        mesh=mesh,
        scratch_types=[
            pltpu.VMEM((H, W), jnp.float32),
            pltpu.VMEM((2, CH), jnp.float32),
            pltpu.VMEM((2, CH), jnp.float32),
            pltpu.VMEM((2, CH), jnp.float32),
            pltpu.VMEM((2, 4, CH), jnp.int32),
            pltpu.VMEM((2, 4, CH, C), jnp.float32),
            pltpu.VMEM((2, CH), jnp.float32),
            pltpu.VMEM((2, CH), jnp.float32),
            pltpu.VMEM((2, CH), jnp.float32),
            pltpu.VMEM((2, C // 8, 8, CH), jnp.float32),
            pltpu.SemaphoreType.DMA((2,)),
            pltpu.SemaphoreType.DMA((2,)),
            pltpu.SemaphoreType.DMA((2,)),
        ],
        compiler_params=pltpu.CompilerParams(needs_layout_passes=False,
                                             use_tc_tiling_on_sc=False),
        name="mv_grid_sample_sc",
    )(x, y, z, table, depth_t)
    outf = outf.transpose(0, 2, 4, 1, 3).reshape(N, P, C)
    return outf, outz.reshape(N, P, 1)


# speedup vs baseline: 2.3408x; 1.0012x over previous
import functools

import jax
import jax.numpy as jnp
from jax import lax
from jax.experimental import pallas as pl
from jax.experimental.pallas import tpu as pltpu
from jax.experimental.pallas import tpu_sc as plsc


def _sc_body(N, C, H, W, P, WPI, PW, CH, NCH,
             x_hbm, y_hbm, z_hbm, table_hbm, depth_hbm, outf_hbm, outz_hbm,
             depth_v, x_v, y_v, z_v, idx_v, rows_v, wx_v, wy_v, dz_v, out_v,
             sem_in, sem_g, sem_out):
    cid = lax.axis_index("c")
    sid = lax.axis_index("s")
    nc = lax.axis_size("c")
    wid = sid * nc + cid
    n = wid // WPI
    part = wid % WPI
    pbase = part * PW
    nbase = n * (H * W)

    pltpu.sync_copy(depth_hbm.at[n], depth_v)

    def in_pairs(b, i):
        sl = pl.ds(pbase + i * CH, CH)
        return ((x_hbm.at[n, sl], x_v.at[b]),
                (y_hbm.at[n, sl], y_v.at[b]),
                (z_hbm.at[n, sl], z_v.at[b]))

    def fire_in(b, i):
        for src, dst in in_pairs(b, i):
            pltpu.async_copy(src, dst, sem_in.at[b])

    def wait_in(b, i):
        for src, dst in in_pairs(b, i):
            pltpu.make_async_copy(src, dst, sem_in.at[b]).wait()

    def fire_gather(b):
        for t in range(4):
            pltpu.async_copy(table_hbm.at[idx_v.at[b, t]], rows_v.at[b, t],
                             sem_g.at[b])

    def wait_gather(b):
        for t in range(4):
            pltpu.make_async_copy(table_hbm.at[idx_v.at[b, t]],
                                  rows_v.at[b, t], sem_g.at[b]).wait()

    def out_dsts(b, i):
        off = pbase + i * CH
        pb = off // CH
        return ((out_v.at[b], outf_hbm.at[n, :, pb]),
                (dz_v.at[b], outz_hbm.at[n, pl.ds(off, CH)]))

    def fire_out(b, i):
        for src, dst in out_dsts(b, i):
            pltpu.async_copy(src, dst, sem_out.at[b])

    def wait_out(b, i):
        for src, dst in out_dsts(b, i):
            pltpu.make_async_copy(src, dst, sem_out.at[b]).wait()

    def stage(b):
        for v in range(CH // 16):
            sl = pl.ds(v * 16, 16)
            gx = (x_v[b, sl] + 1.0) * ((W - 1) * 0.5)
            gy = (y_v[b, sl] + 1.0) * ((H - 1) * 0.5)
            x0 = jnp.minimum(jnp.maximum(gx.astype(jnp.int32), 0), W - 2)
            y0 = jnp.minimum(jnp.maximum(gy.astype(jnp.int32), 0), H - 2)
            wx = gx - x0.astype(jnp.float32)
            wy = gy - y0.astype(jnp.float32)
            r00 = y0 * W + x0
            g00 = r00 + nbase
            idx_v[b, 0, sl] = g00
            idx_v[b, 1, sl] = g00 + 1
            idx_v[b, 2, sl] = g00 + W
            idx_v[b, 3, sl] = g00 + (W + 1)
            x1 = x0 + 1
            d00 = plsc.load_gather(depth_v, [y0, x0])
            d01 = plsc.load_gather(depth_v, [y0, x1])
            d10 = plsc.load_gather(depth_v, [y0 + 1, x0])
            d11 = plsc.load_gather(depth_v, [y0 + 1, x1])
            dx0 = d00 + wx * (d01 - d00)
            dx1 = d10 + wx * (d11 - d10)
            dd = dx0 + wy * (dx1 - dx0)
            df = z_v[b, sl] - dd
            dz_v[b, sl] = jnp.exp(-200.0 * df * df)
            wx_v[b, sl] = wx
            wy_v[b, sl] = wy

    lane = lax.broadcasted_iota(jnp.int32, (16,), 0)
    cvecs = [(c + lane) & (C - 1) for c in range(C)]
    cbs = [cv >> 3 for cv in cvecs]
    cis = [cv & 7 for cv in cvecs]

    def blend(b):
        @plsc.parallel_loop(0, CH, 16)
        def _pt(p0):
            wxv = wx_v[b, pl.ds(p0, 16)]
            wyv = wy_v[b, pl.ds(p0, 16)]
            pidx = p0 + lane
            for c in range(C):
                v00 = plsc.load_gather(rows_v.at[b, 0], [pidx, cvecs[c]])
                v01 = plsc.load_gather(rows_v.at[b, 1], [pidx, cvecs[c]])
                v10 = plsc.load_gather(rows_v.at[b, 2], [pidx, cvecs[c]])
                v11 = plsc.load_gather(rows_v.at[b, 3], [pidx, cvecs[c]])
                a = v00 + wxv * (v01 - v00)
                bb = v10 + wxv * (v11 - v10)
                res = a + wyv * (bb - a)
                plsc.store_scatter(out_v.at[b], [cbs[c], cis[c], pidx], res)

    fire_in(0, 0)
    fire_in(1, 1)

    def pair_body(k, _):
        for b in (0, 1):
            o = 1 - b
            i = 2 * k + b

            @pl.when(i >= 2)
            def _wait_out_prev():
                wait_out(b, i - 2)

            wait_in(b, i)
            stage(b)
            fire_gather(b)

            @pl.when(i + 2 < NCH)
            def _prefetch():
                fire_in(b, i + 2)

            @pl.when(i >= 1)
            def _blend_prev():
                wait_gather(o)
                blend(o)
                fire_out(o, i - 1)

        return 0

    lax.fori_loop(0, NCH // 2, pair_body, 0)

    last = NCH - 1
    lb = last % 2
    wait_gather(lb)
    blend(lb)
    fire_out(lb, last)
    wait_out(1 - lb, last - 1)
    wait_out(lb, last)


def kernel(rgbd_feats, rgbs, depths, proj_xy, proj_z, num_views):
    del rgbs, num_views
    N, C, H, W = rgbd_feats.shape
    B = proj_xy.shape[0]
    P = proj_xy.shape[2] * proj_xy.shape[3] * proj_xy.shape[4]
    assert N == B * proj_xy.shape[1]

    info = plsc.get_sparse_core_info()
    NW = info.num_cores * info.num_subcores
    WPI = NW // N
    PW = P // WPI
    CH = 128
    NCH = PW // CH
    assert P % WPI == 0 and PW % CH == 0 and NCH % 2 == 0

    def _tr_body(in_ref, out_ref, tmp_ref):
        rows = W * C // 128
        for yy in range(8):
            tmp_ref[...] = jnp.transpose(in_ref[:, yy, :], (1, 0))
            for k in range(4):
                sub = tmp_ref[pl.Slice(k, W // 4, 4), :]
                out_ref[yy * rows:(yy + 1) * rows, k * C:(k + 1) * C] = sub

    table128 = pl.pallas_call(
        _tr_body,
        grid=(N, H // 8),
        in_specs=[pl.BlockSpec((C, 8, W), lambda n, y: (n, y, 0))],
        out_specs=pl.BlockSpec((8 * W * C // 128, 128),
                               lambda n, y: (n * (H // 8) + y, 0)),
        out_shape=jax.ShapeDtypeStruct((N * H * W * C // 128, 128),
                                       jnp.float32),
        scratch_shapes=[pltpu.VMEM((W, C), jnp.float32)],
    )(rgbd_feats.reshape(N * C, H, W))
    table = table128.reshape(N * H * W, C)

    depth_t = depths.reshape(N, H, W)
    x = proj_xy[..., 0].reshape(N, P)
    y = proj_xy[..., 1].reshape(N, P)
    z = proj_z.reshape(N, P)

    mesh = plsc.VectorSubcoreMesh(core_axis_name="c", subcore_axis_name="s")
    body = functools.partial(_sc_body, N, C, H, W, P, WPI, PW, CH, NCH)
    outf, outz = pl.kernel(
        body,
        out_type=(
            jax.ShapeDtypeStruct((N, C // 8, P // CH, 8, CH), jnp.float32),
            jax.ShapeDtypeStruct((N, P), jnp.float32),
        ),
        mesh=mesh,
        scratch_types=[
            pltpu.VMEM((H, W), jnp.float32),
            pltpu.VMEM((2, CH), jnp.float32),
            pltpu.VMEM((2, CH), jnp.float32),
            pltpu.VMEM((2, CH), jnp.float32),
            pltpu.VMEM((2, 4, CH), jnp.int32),
            pltpu.VMEM((2, 4, CH, C), jnp.float32),
            pltpu.VMEM((2, CH), jnp.float32),
            pltpu.VMEM((2, CH), jnp.float32),
            pltpu.VMEM((2, CH), jnp.float32),
            pltpu.VMEM((2, C // 8, 8, CH), jnp.float32),
            pltpu.SemaphoreType.DMA((2,)),
            pltpu.SemaphoreType.DMA((2,)),
            pltpu.SemaphoreType.DMA((2,)),
        ],
        compiler_params=pltpu.CompilerParams(needs_layout_passes=False,
                                             use_tc_tiling_on_sc=False),
        name="mv_grid_sample_sc",
    )(x, y, z, table, depth_t)
    outf = outf.transpose(0, 2, 4, 1, 3).reshape(N, P, C)
    return outf, outz.reshape(N, P, 1)
